# Initial kernel scaffold; baseline (speedup 1.0000x reference)
import jax
import jax.numpy as jnp
from jax.experimental import pallas as pl


def kernel(x, edge_index, batch, W1, b1, W2, b2, W3, b3, W4, b4, f1W, f1b, f2W, f2b):
    raise NotImplementedError("write your pallas kernel here")


# speedup vs baseline: 4.1651x; 4.1651x over previous
import functools

import jax
import jax.numpy as jnp
from jax import lax
from jax.experimental import pallas as pl
from jax.experimental.pallas import tpu as pltpu
from jax.experimental.pallas import tpu_sc as plsc

NC = 2
NS = 16
B = 128
CW = 32

_MESH = plsc.VectorSubcoreMesh(
    core_axis_name="c", subcore_axis_name="s", num_cores=NC, num_subcores=NS)


def _fill_zeros_1d(ref, n):
    def body(i, _):
        ref[pl.ds(i * 16, 16)] = jnp.zeros((16,), jnp.float32)
        return 0
    lax.fori_loop(0, n // 16, body, 0)


def _fill_zeros_2d(ref, nrows):
    def body(i, _):
        ref[i, pl.ds(0, 16)] = jnp.zeros((16,), jnp.float32)
        ref[i, pl.ds(16, 16)] = jnp.zeros((16,), jnp.float32)
        return 0
    lax.fori_loop(0, nrows, body, 0)


def _make_deg(NB, Npad):
    SP = Npad // NS

    @functools.partial(
        pl.kernel,
        out_type=(jax.ShapeDtypeStruct((Npad,), jnp.float32),
                  jax.ShapeDtypeStruct((Npad,), jnp.float32)),
        mesh=_MESH,
        compiler_params=pltpu.CompilerParams(use_tc_tiling_on_sc=False),
        scratch_types=[
            pltpu.VMEM((B,), jnp.int32),
            pltpu.VMEM((B,), jnp.float32),
            pltpu.VMEM((SP,), jnp.float32),
            pltpu.VMEM_SHARED((Npad,), jnp.float32),
        ],
    )
    def deg_kernel(dst_e, deg_a, deg_b, dst_v, ones_v, zer_v, acc_sh):
        c = lax.axis_index("c")
        t = lax.axis_index("s")
        w = c * NS + t
        for k in range(B // 16):
            ones_v[pl.ds(k * 16, 16)] = jnp.full((16,), 1.0, jnp.float32)
        _fill_zeros_1d(zer_v, SP)
        pltpu.sync_copy(zer_v, acc_sh.at[pl.ds(t * SP, SP)])
        plsc.subcore_barrier()
        nw = NC * NS
        nb = NB // nw + jnp.where(w < NB % nw, 1, 0)

        def body(i, _):
            j = w + i * nw
            pltpu.sync_copy(dst_e.at[pl.ds(j * B, B)], dst_v)
            pltpu.sync_copy(ones_v, acc_sh.at[dst_v], add=True)
            return 0

        lax.fori_loop(0, nb, body, 0)
        plsc.subcore_barrier()

        @pl.when(c == 0)
        def _():
            pltpu.sync_copy(acc_sh.at[pl.ds(t * SP, SP)],
                            deg_a.at[pl.ds(t * SP, SP)])

        @pl.when(c == 1)
        def _():
            pltpu.sync_copy(acc_sh.at[pl.ds(t * SP, SP)],
                            deg_b.at[pl.ds(t * SP, SP)])

    return deg_kernel


def _make_spmv(NB, Npad):
    SP = Npad // NS

    @functools.partial(
        pl.kernel,
        out_type=(jax.ShapeDtypeStruct((Npad,), jnp.float32),
                  jax.ShapeDtypeStruct((Npad,), jnp.float32)),
        mesh=_MESH,
        compiler_params=pltpu.CompilerParams(use_tc_tiling_on_sc=False),
        scratch_types=[
            pltpu.VMEM((B,), jnp.int32),
            pltpu.VMEM((B,), jnp.int32),
            pltpu.VMEM((B,), jnp.float32),
            pltpu.VMEM((SP,), jnp.float32),
            pltpu.VMEM_SHARED((Npad,), jnp.float32),
            pltpu.SemaphoreType.DMA,
        ],
    )
    def spmv_kernel(xp, src_e, dst_e, s_a, s_b,
                    src_v, dst_v, vals_v, zer_v, acc_sh, sem):
        c = lax.axis_index("c")
        t = lax.axis_index("s")
        w = c * NS + t
        _fill_zeros_1d(zer_v, SP)
        pltpu.sync_copy(zer_v, acc_sh.at[pl.ds(t * SP, SP)])
        plsc.subcore_barrier()
        nw = NC * NS
        nb = NB // nw + jnp.where(w < NB % nw, 1, 0)

        def body(i, _):
            j = w + i * nw
            pltpu.sync_copy(src_e.at[pl.ds(j * B, B)], src_v)
            pltpu.sync_copy(dst_e.at[pl.ds(j * B, B)], dst_v)
            pltpu.async_copy(xp.at[src_v], vals_v, sem).wait()
            pltpu.sync_copy(vals_v, acc_sh.at[dst_v], add=True)
            return 0

        lax.fori_loop(0, nb, body, 0)
        plsc.subcore_barrier()

        @pl.when(c == 0)
        def _():
            pltpu.sync_copy(acc_sh.at[pl.ds(t * SP, SP)],
                            s_a.at[pl.ds(t * SP, SP)])

        @pl.when(c == 1)
        def _():
            pltpu.sync_copy(acc_sh.at[pl.ds(t * SP, SP)],
                            s_b.at[pl.ds(t * SP, SP)])

    return spmv_kernel


def _make_spmm(NB, Npad, HG, nacc):
    SPA = nacc // NS
    SP4 = SPA // 4
    GPC = HG // NC

    @functools.partial(
        pl.kernel,
        out_type=jax.ShapeDtypeStruct((HG, Npad, CW), jnp.float32),
        mesh=_MESH,
        compiler_params=pltpu.CompilerParams(use_tc_tiling_on_sc=False),
        scratch_types=[
            pltpu.VMEM((B,), jnp.int32),
            pltpu.VMEM((B,), jnp.int32),
            pltpu.VMEM((B, CW), jnp.float32),
            pltpu.VMEM((SP4, CW), jnp.float32),
            pltpu.VMEM_SHARED((nacc, CW), jnp.float32),
            pltpu.SemaphoreType.DMA,
        ],
    )
    def spmm_kernel(z, src_e, dst_e, s_out,
                    src_v, dst_v, rows_v, zer_v, acc_sh, sem):
        c = lax.axis_index("c")
        t = lax.axis_index("s")
        _fill_zeros_2d(zer_v, SP4)
        nb = NB // NS + jnp.where(t < NB % NS, 1, 0)
        for jg in range(GPC):
            g = c * GPC + jg
            off = g * Npad
            for q in range(4):
                pltpu.sync_copy(zer_v, acc_sh.at[pl.ds(t * SPA + q * SP4, SP4)])
            plsc.subcore_barrier()

            def body(i, _):
                j = t + i * NS
                pltpu.sync_copy(src_e.at[pl.ds(j * B, B)], src_v)
                pltpu.sync_copy(dst_e.at[pl.ds(j * B, B)], dst_v)
                for k in range(B // 16):
                    src_v[pl.ds(k * 16, 16)] = src_v[pl.ds(k * 16, 16)] + off
                pltpu.async_copy(z.at[src_v], rows_v, sem).wait()
                pltpu.sync_copy(rows_v, acc_sh.at[dst_v], add=True)
                return 0

            lax.fori_loop(0, nb, body, 0)
            plsc.subcore_barrier()
            pltpu.sync_copy(acc_sh.at[pl.ds(t * SPA, SPA)],
                            s_out.at[g, pl.ds(t * SPA, SPA)])

    return spmm_kernel


def _make_l1_body(HG):
    def l1_body(s2a_ref, s2b_ref, xp_ref, dis_ref, w1_ref, b1_ref, w2_ref,
                out_ref, hs_ref):
        go = pl.program_id(1)

        @pl.when(go == 0)
        def _():
            sv = dis_ref[...] * (s2a_ref[...] + s2b_ref[...] + xp_ref[...])
            hs_ref[...] = jnp.maximum(sv * w1_ref[...] + b1_ref[...], 0.0)

        out_ref[0] = jnp.dot(hs_ref[...], w2_ref[0],
                             preferred_element_type=jnp.float32) * dis_ref[...]

    return l1_body


def _make_mid_body(HG):
    def mid_body(s_ref, z_ref, dis_ref, b_ref, w_ref, out_ref, hs_ref):
        go = pl.program_id(1)

        @pl.when(go == 0)
        def _():
            dis = dis_ref[...]
            for gi in range(HG):
                hs_ref[:, pl.ds(gi * CW, CW)] = jnp.maximum(
                    dis * (s_ref[gi] + z_ref[gi]) + b_ref[gi], 0.0)

        out_ref[0] = jnp.dot(hs_ref[...], w_ref[0],
                             preferred_element_type=jnp.float32) * dis_ref[...]

    return mid_body


def _make_fin_body(R, NBLK, HG, n_real):
    def fin_body(s_ref, z_ref, dis_ref, b_ref, f1w_ref, f1b_ref, f2w_ref,
                 f2b_ref, out_ref, acc_ref):
        i = pl.program_id(0)
        dis = dis_ref[...]
        rows = i * R + lax.broadcasted_iota(jnp.int32, (R, 1), 0)
        mask = rows < n_real
        for gi in range(HG):
            h = jnp.maximum(dis * (s_ref[gi] + z_ref[gi]) + b_ref[gi], 0.0)
            h = jnp.where(mask, h, 0.0)
            part = jnp.sum(h, axis=0, keepdims=True)

            @pl.when(i == 0)
            def _():
                acc_ref[:, pl.ds(gi * CW, CW)] = part

            @pl.when(i > 0)
            def _():
                acc_ref[:, pl.ds(gi * CW, CW)] = (
                    acc_ref[:, pl.ds(gi * CW, CW)] + part)

        @pl.when(i == NBLK - 1)
        def _():
            g = acc_ref[...] * (1.0 / n_real)
            o = jnp.maximum(
                jnp.dot(g, f1w_ref[...], preferred_element_type=jnp.float32)
                + f1b_ref[...], 0.0)
            out_ref[...] = (jnp.dot(o, f2w_ref[...],
                                    preferred_element_type=jnp.float32)
                            + f2b_ref[...])

    return fin_body


def _prep_body(dega_ref, degb_ref, x_ref, dis_ref, xp_ref):
    d = dega_ref[...] + degb_ref[...] + 1.0
    dis = lax.rsqrt(d)
    dis_ref[...] = dis
    xp_ref[...] = dis * x_ref[...]


def kernel(x, edge_index, batch, W1, b1, W2, b2, W3, b3, W4, b4,
           f1W, f1b, f2W, f2b):
    N = x.shape[0]
    E = edge_index.shape[1]
    H = W2.shape[0]
    HG = H // CW
    Npad = ((N + NS * B - 1) // (NS * B)) * (NS * B)
    NB = E // B
    R = 1600
    NBLK = Npad // R

    src_e = edge_index[0]
    dst_e = edge_index[1]
    x_pad = jnp.concatenate(
        [x, jnp.zeros((Npad - N, 1), jnp.float32)], axis=0)

    deg_a, deg_b = _make_deg(NB, Npad)(dst_e)

    vspec1 = pl.BlockSpec((R, 1), lambda i: (i, 0))
    dis, xp = pl.pallas_call(
        _prep_body,
        grid=(NBLK,),
        in_specs=[vspec1, vspec1, vspec1],
        out_specs=(vspec1, vspec1),
        out_shape=(jax.ShapeDtypeStruct((Npad, 1), jnp.float32),
                   jax.ShapeDtypeStruct((Npad, 1), jnp.float32)),
    )(deg_a.reshape(Npad, 1), deg_b.reshape(Npad, 1), x_pad)

    s2a, s2b = _make_spmv(NB, Npad)(xp.reshape(Npad), src_e, dst_e)

    grp_spec = pl.BlockSpec((HG, R, CW), lambda i, go: (0, i, 0))
    vec_spec2 = pl.BlockSpec((R, 1), lambda i, go: (i, 0))
    wcol_spec = pl.BlockSpec((1, H, CW), lambda i, go: (go, 0, 0))
    b8_spec = pl.BlockSpec((HG, 1, CW), lambda i, go: (0, 0, 0))
    out_spec = pl.BlockSpec((1, R, CW), lambda i, go: (go, i, 0))
    zg_shape = jax.ShapeDtypeStruct((HG, Npad, CW), jnp.float32)

    def split_w(W):
        return W.reshape(H, HG, CW).transpose(1, 0, 2)

    def split_b(b):
        return b.reshape(HG, 1, CW)

    Z2 = pl.pallas_call(
        _make_l1_body(HG),
        grid=(NBLK, HG),
        in_specs=[vec_spec2, vec_spec2, vec_spec2, vec_spec2,
                  pl.BlockSpec((1, H), lambda i, go: (0, 0)),
                  pl.BlockSpec((1, H), lambda i, go: (0, 0)),
                  wcol_spec],
        out_specs=out_spec,
        out_shape=zg_shape,
        scratch_shapes=[pltpu.VMEM((R, H), jnp.float32)],
    )(s2a.reshape(Npad, 1), s2b.reshape(Npad, 1), xp, dis,
      W1, b1.reshape(1, H), split_w(W2))

    nacc = ((N + NS * 8 - 1) // (NS * 8)) * (NS * 8)
    spmm = _make_spmm(NB, Npad, HG, nacc)

    def mid(S, Z, bprev, Wnext):
        return pl.pallas_call(
            _make_mid_body(HG),
            grid=(NBLK, HG),
            in_specs=[grp_spec, grp_spec, vec_spec2, b8_spec, wcol_spec],
            out_specs=out_spec,
            out_shape=zg_shape,
            scratch_shapes=[pltpu.VMEM((R, H), jnp.float32)],
        )(S, Z, dis, split_b(bprev), split_w(Wnext))

    S2 = spmm(Z2.reshape(HG * Npad, CW), src_e, dst_e)
    Z3 = mid(S2, Z2, b2, W3)
    S3 = spmm(Z3.reshape(HG * Npad, CW), src_e, dst_e)
    Z4 = mid(S3, Z3, b3, W4)
    S4 = spmm(Z4.reshape(HG * Npad, CW), src_e, dst_e)

    y = pl.pallas_call(
        _make_fin_body(R, NBLK, HG, N),
        grid=(NBLK,),
        in_specs=[pl.BlockSpec((HG, R, CW), lambda i: (0, i, 0)),
                  pl.BlockSpec((HG, R, CW), lambda i: (0, i, 0)),
                  pl.BlockSpec((R, 1), lambda i: (i, 0)),
                  pl.BlockSpec((HG, 1, CW), lambda i: (0, 0, 0)),
                  pl.BlockSpec((H, H), lambda i: (0, 0)),
                  pl.BlockSpec((1, H), lambda i: (0, 0)),
                  pl.BlockSpec((H, 1), lambda i: (0, 0)),
                  pl.BlockSpec((1, 1), lambda i: (0, 0))],
        out_specs=pl.BlockSpec((1, 1), lambda i: (0, 0)),
        out_shape=jax.ShapeDtypeStruct((1, 1), jnp.float32),
        scratch_shapes=[pltpu.VMEM((1, H), jnp.float32)],
    )(S4, Z4, dis, split_b(b4), f1W, f1b.reshape(1, H),
      f2W, f2b.reshape(1, 1))

    return y


# speedup vs baseline: 10.8015x; 2.5933x over previous
import functools

import jax
import jax.numpy as jnp
from jax import lax
from jax.experimental import pallas as pl
from jax.experimental.pallas import tpu as pltpu
from jax.experimental.pallas import tpu_sc as plsc

NC = 2
NS = 16
B = 128
CW = 32

_MESH = plsc.VectorSubcoreMesh(
    core_axis_name="c", subcore_axis_name="s", num_cores=NC, num_subcores=NS)


def _fill_zeros_1d(ref, n):
    def body(i, _):
        ref[pl.ds(i * 16, 16)] = jnp.zeros((16,), jnp.float32)
        return 0
    lax.fori_loop(0, n // 16, body, 0)


def _fill_zeros_2d(ref, nrows):
    def body(i, _):
        ref[i, pl.ds(0, 16)] = jnp.zeros((16,), jnp.float32)
        ref[i, pl.ds(16, 16)] = jnp.zeros((16,), jnp.float32)
        return 0
    lax.fori_loop(0, nrows, body, 0)


def _make_deg(NB, Npad):
    SP = Npad // NS

    @functools.partial(
        pl.kernel,
        out_type=(jax.ShapeDtypeStruct((Npad,), jnp.float32),
                  jax.ShapeDtypeStruct((Npad,), jnp.float32)),
        mesh=_MESH,
        compiler_params=pltpu.CompilerParams(use_tc_tiling_on_sc=False),
        scratch_types=[
            pltpu.VMEM((B,), jnp.int32),
            pltpu.VMEM((B,), jnp.float32),
            pltpu.VMEM((SP,), jnp.float32),
            pltpu.VMEM_SHARED((Npad,), jnp.float32),
        ],
    )
    def deg_kernel(dst_e, deg_a, deg_b, dst_v, ones_v, zer_v, acc_sh):
        c = lax.axis_index("c")
        t = lax.axis_index("s")
        w = c * NS + t
        for k in range(B // 16):
            ones_v[pl.ds(k * 16, 16)] = jnp.full((16,), 1.0, jnp.float32)
        _fill_zeros_1d(zer_v, SP)
        pltpu.sync_copy(zer_v, acc_sh.at[pl.ds(t * SP, SP)])
        plsc.subcore_barrier()
        nw = NC * NS
        nb = NB // nw + jnp.where(w < NB % nw, 1, 0)

        def body(i, _):
            j = w + i * nw
            pltpu.sync_copy(dst_e.at[pl.ds(j * B, B)], dst_v)
            pltpu.sync_copy(ones_v, acc_sh.at[dst_v], add=True)
            return 0

        lax.fori_loop(0, nb, body, 0)
        plsc.subcore_barrier()

        @pl.when(c == 0)
        def _():
            pltpu.sync_copy(acc_sh.at[pl.ds(t * SP, SP)],
                            deg_a.at[pl.ds(t * SP, SP)])

        @pl.when(c == 1)
        def _():
            pltpu.sync_copy(acc_sh.at[pl.ds(t * SP, SP)],
                            deg_b.at[pl.ds(t * SP, SP)])

    return deg_kernel


def _make_spmv(NB, Npad):
    SP = Npad // NS

    @functools.partial(
        pl.kernel,
        out_type=(jax.ShapeDtypeStruct((Npad,), jnp.float32),
                  jax.ShapeDtypeStruct((Npad,), jnp.float32)),
        mesh=_MESH,
        compiler_params=pltpu.CompilerParams(use_tc_tiling_on_sc=False),
        scratch_types=[
            pltpu.VMEM((B,), jnp.int32),
            pltpu.VMEM((B,), jnp.int32),
            pltpu.VMEM((B,), jnp.float32),
            pltpu.VMEM((SP,), jnp.float32),
            pltpu.VMEM_SHARED((Npad,), jnp.float32),
            pltpu.SemaphoreType.DMA,
        ],
    )
    def spmv_kernel(xp, src_e, dst_e, s_a, s_b,
                    src_v, dst_v, vals_v, zer_v, acc_sh, sem):
        c = lax.axis_index("c")
        t = lax.axis_index("s")
        w = c * NS + t
        _fill_zeros_1d(zer_v, SP)
        pltpu.sync_copy(zer_v, acc_sh.at[pl.ds(t * SP, SP)])
        plsc.subcore_barrier()
        nw = NC * NS
        nb = NB // nw + jnp.where(w < NB % nw, 1, 0)

        def body(i, _):
            j = w + i * nw
            pltpu.sync_copy(src_e.at[pl.ds(j * B, B)], src_v)
            pltpu.sync_copy(dst_e.at[pl.ds(j * B, B)], dst_v)
            pltpu.async_copy(xp.at[src_v], vals_v, sem).wait()
            pltpu.sync_copy(vals_v, acc_sh.at[dst_v], add=True)
            return 0

        lax.fori_loop(0, nb, body, 0)
        plsc.subcore_barrier()

        @pl.when(c == 0)
        def _():
            pltpu.sync_copy(acc_sh.at[pl.ds(t * SP, SP)],
                            s_a.at[pl.ds(t * SP, SP)])

        @pl.when(c == 1)
        def _():
            pltpu.sync_copy(acc_sh.at[pl.ds(t * SP, SP)],
                            s_b.at[pl.ds(t * SP, SP)])

    return spmv_kernel


NBUF = 4
CH = 28
ZR = 184


def _make_spmm(NB2, Npad, HG, nacc):
    SPA = nacc // NS
    NZC = SPA // ZR
    GPC = HG // NC
    TPB = NB2 // NS
    NCH = TPB // CH
    MS = (CH - NBUF) // NBUF

    @functools.partial(
        pl.kernel,
        out_type=jax.ShapeDtypeStruct((HG, Npad, CW), jnp.float32),
        mesh=_MESH,
        compiler_params=pltpu.CompilerParams(use_tc_tiling_on_sc=False),
        scratch_types=[
            pltpu.VMEM((CH, B), jnp.int32),
            pltpu.VMEM((CH, B), jnp.int32),
            [pltpu.VMEM((B, CW), jnp.float32) for _ in range(NBUF)],
            pltpu.VMEM((ZR, CW), jnp.float32),
            pltpu.VMEM_SHARED((nacc, CW), jnp.float32),
            [pltpu.SemaphoreType.DMA for _ in range(NBUF)],
        ],
    )
    def spmm_kernel(z, src2, dst2, s_out,
                    srcc_v, dstc_v, rows_l, zer_v, acc_sh, sem_l):
        c = lax.axis_index("c")
        t = lax.axis_index("s")
        _fill_zeros_2d(zer_v, ZR)
        b0 = t * TPB
        for jg in range(GPC):
            g = c * GPC + jg
            off = g * Npad
            def zero_body(q4, _):
                pltpu.sync_copy(zer_v, acc_sh.at[pl.ds(t * SPA + q4 * ZR, ZR)])
                return 0
            lax.fori_loop(0, NZC, zero_body, 0)
            plsc.subcore_barrier()

            def chunk_body(ci, _):
                cb = b0 + ci * CH
                pltpu.sync_copy(src2.at[pl.ds(cb, CH)], srcc_v)
                pltpu.sync_copy(dst2.at[pl.ds(cb, CH)], dstc_v)

                def add_off(r, _):
                    for k in range(B // 16):
                        srcc_v[r, pl.ds(k * 16, 16)] = (
                            srcc_v[r, pl.ds(k * 16, 16)] + off)
                    return 0

                lax.fori_loop(0, CH, add_off, 0)
                for q in range(NBUF):
                    pltpu.async_copy(z.at[srcc_v.at[q]], rows_l[q], sem_l[q])

                def ring_body(i2, _):
                    for q in range(NBUF):
                        s = i2 * NBUF + q
                        pltpu.make_async_copy(
                            z.at[srcc_v.at[s]], rows_l[q], sem_l[q]).wait()
                        pltpu.sync_copy(rows_l[q], acc_sh.at[dstc_v.at[s]],
                                        add=True)
                        pltpu.async_copy(z.at[srcc_v.at[s + NBUF]],
                                         rows_l[q], sem_l[q])
                    return 0

                lax.fori_loop(0, MS, ring_body, 0)
                for q in range(NBUF):
                    s = MS * NBUF + q
                    pltpu.make_async_copy(
                        z.at[srcc_v.at[s]], rows_l[q], sem_l[q]).wait()
                    pltpu.sync_copy(rows_l[q], acc_sh.at[dstc_v.at[s]],
                                    add=True)
                return 0

            lax.fori_loop(0, NCH, chunk_body, 0)
            plsc.subcore_barrier()
            pltpu.sync_copy(acc_sh.at[pl.ds(t * SPA, SPA)],
                            s_out.at[g, pl.ds(t * SPA, SPA)])

    return spmm_kernel


def _make_l1_body(HG):
    def l1_body(s2a_ref, s2b_ref, xp_ref, dis_ref, w1_ref, b1_ref, w2_ref,
                out_ref, hs_ref):
        go = pl.program_id(1)

        @pl.when(go == 0)
        def _():
            sv = dis_ref[...] * (s2a_ref[...] + s2b_ref[...] + xp_ref[...])
            hs_ref[...] = jnp.maximum(sv * w1_ref[...] + b1_ref[...], 0.0)

        out_ref[0] = jnp.dot(hs_ref[...], w2_ref[0],
                             preferred_element_type=jnp.float32) * dis_ref[...]

    return l1_body


def _make_mid_body(HG):
    def mid_body(s_ref, z_ref, dis_ref, b_ref, w_ref, out_ref, hs_ref):
        go = pl.program_id(1)

        @pl.when(go == 0)
        def _():
            dis = dis_ref[...]
            for gi in range(HG):
                hs_ref[:, pl.ds(gi * CW, CW)] = jnp.maximum(
                    dis * (s_ref[gi] + z_ref[gi]) + b_ref[gi], 0.0)

        out_ref[0] = jnp.dot(hs_ref[...], w_ref[0],
                             preferred_element_type=jnp.float32) * dis_ref[...]

    return mid_body


def _make_fin_body(R, NBLK, HG, n_real):
    def fin_body(s_ref, z_ref, dis_ref, b_ref, f1w_ref, f1b_ref, f2w_ref,
                 f2b_ref, out_ref, acc_ref):
        i = pl.program_id(0)
        dis = dis_ref[...]
        rows = i * R + lax.broadcasted_iota(jnp.int32, (R, 1), 0)
        mask = rows < n_real
        for gi in range(HG):
            h = jnp.maximum(dis * (s_ref[gi] + z_ref[gi]) + b_ref[gi], 0.0)
            h = jnp.where(mask, h, 0.0)
            part = jnp.sum(h, axis=0, keepdims=True)

            @pl.when(i == 0)
            def _():
                acc_ref[:, pl.ds(gi * CW, CW)] = part

            @pl.when(i > 0)
            def _():
                acc_ref[:, pl.ds(gi * CW, CW)] = (
                    acc_ref[:, pl.ds(gi * CW, CW)] + part)

        @pl.when(i == NBLK - 1)
        def _():
            g = acc_ref[...] * (1.0 / n_real)
            o = jnp.maximum(
                jnp.dot(g, f1w_ref[...], preferred_element_type=jnp.float32)
                + f1b_ref[...], 0.0)
            out_ref[...] = (jnp.dot(o, f2w_ref[...],
                                    preferred_element_type=jnp.float32)
                            + f2b_ref[...])

    return fin_body


def _prep_body(dega_ref, degb_ref, x_ref, dis_ref, xp_ref):
    d = dega_ref[...] + degb_ref[...] + 1.0
    dis = lax.rsqrt(d)
    dis_ref[...] = dis
    xp_ref[...] = dis * x_ref[...]


def kernel(x, edge_index, batch, W1, b1, W2, b2, W3, b3, W4, b4,
           f1W, f1b, f2W, f2b):
    N = x.shape[0]
    E = edge_index.shape[1]
    H = W2.shape[0]
    HG = H // CW
    Npad = ((N + NS * B - 1) // (NS * B)) * (NS * B)
    NB = E // B
    R = 1600
    NBLK = Npad // R

    nacc = ((N + NS * 8 - 1) // (NS * 8)) * (NS * 8)
    nw = NC * NS
    NB2 = ((NB + nw - 1) // nw) * nw
    EP = NB2 * B - E
    src_e = jnp.concatenate(
        [edge_index[0], (jnp.arange(EP, dtype=jnp.int32) % N)])
    dst_e = jnp.concatenate(
        [edge_index[1], N + (jnp.arange(EP, dtype=jnp.int32) % (nacc - N))])
    src2 = src_e.reshape(NB2, B)
    dst2 = dst_e.reshape(NB2, B)
    x_pad = jnp.concatenate(
        [x, jnp.zeros((Npad - N, 1), jnp.float32)], axis=0)

    deg_a, deg_b = _make_deg(NB2, Npad)(dst_e)

    vspec1 = pl.BlockSpec((R, 1), lambda i: (i, 0))
    dis, xp = pl.pallas_call(
        _prep_body,
        grid=(NBLK,),
        in_specs=[vspec1, vspec1, vspec1],
        out_specs=(vspec1, vspec1),
        out_shape=(jax.ShapeDtypeStruct((Npad, 1), jnp.float32),
                   jax.ShapeDtypeStruct((Npad, 1), jnp.float32)),
    )(deg_a.reshape(Npad, 1), deg_b.reshape(Npad, 1), x_pad)

    s2a, s2b = _make_spmv(NB2, Npad)(xp.reshape(Npad), src_e, dst_e)

    grp_spec = pl.BlockSpec((HG, R, CW), lambda i, go: (0, i, 0))
    vec_spec2 = pl.BlockSpec((R, 1), lambda i, go: (i, 0))
    wcol_spec = pl.BlockSpec((1, H, CW), lambda i, go: (go, 0, 0))
    b8_spec = pl.BlockSpec((HG, 1, CW), lambda i, go: (0, 0, 0))
    out_spec = pl.BlockSpec((1, R, CW), lambda i, go: (go, i, 0))
    zg_shape = jax.ShapeDtypeStruct((HG, Npad, CW), jnp.float32)

    def split_w(W):
        return W.reshape(H, HG, CW).transpose(1, 0, 2)

    def split_b(b):
        return b.reshape(HG, 1, CW)

    Z2 = pl.pallas_call(
        _make_l1_body(HG),
        grid=(NBLK, HG),
        in_specs=[vec_spec2, vec_spec2, vec_spec2, vec_spec2,
                  pl.BlockSpec((1, H), lambda i, go: (0, 0)),
                  pl.BlockSpec((1, H), lambda i, go: (0, 0)),
                  wcol_spec],
        out_specs=out_spec,
        out_shape=zg_shape,
        scratch_shapes=[pltpu.VMEM((R, H), jnp.float32)],
    )(s2a.reshape(Npad, 1), s2b.reshape(Npad, 1), xp, dis,
      W1, b1.reshape(1, H), split_w(W2))

    spmm = _make_spmm(NB2, Npad, HG, nacc)

    def mid(S, Z, bprev, Wnext):
        return pl.pallas_call(
            _make_mid_body(HG),
            grid=(NBLK, HG),
            in_specs=[grp_spec, grp_spec, vec_spec2, b8_spec, wcol_spec],
            out_specs=out_spec,
            out_shape=zg_shape,
            scratch_shapes=[pltpu.VMEM((R, H), jnp.float32)],
        )(S, Z, dis, split_b(bprev), split_w(Wnext))

    S2 = spmm(Z2.reshape(HG * Npad, CW), src2, dst2)
    Z3 = mid(S2, Z2, b2, W3)
    S3 = spmm(Z3.reshape(HG * Npad, CW), src2, dst2)
    Z4 = mid(S3, Z3, b3, W4)
    S4 = spmm(Z4.reshape(HG * Npad, CW), src2, dst2)

    y = pl.pallas_call(
        _make_fin_body(R, NBLK, HG, N),
        grid=(NBLK,),
        in_specs=[pl.BlockSpec((HG, R, CW), lambda i: (0, i, 0)),
                  pl.BlockSpec((HG, R, CW), lambda i: (0, i, 0)),
                  pl.BlockSpec((R, 1), lambda i: (i, 0)),
                  pl.BlockSpec((HG, 1, CW), lambda i: (0, 0, 0)),
                  pl.BlockSpec((H, H), lambda i: (0, 0)),
                  pl.BlockSpec((1, H), lambda i: (0, 0)),
                  pl.BlockSpec((H, 1), lambda i: (0, 0)),
                  pl.BlockSpec((1, 1), lambda i: (0, 0))],
        out_specs=pl.BlockSpec((1, 1), lambda i: (0, 0)),
        out_shape=jax.ShapeDtypeStruct((1, 1), jnp.float32),
        scratch_shapes=[pltpu.VMEM((1, H), jnp.float32)],
    )(S4, Z4, dis, split_b(b4), f1W, f1b.reshape(1, H),
      f2W, f2b.reshape(1, 1))

    return y


# speedup vs baseline: 11.5385x; 1.0682x over previous
import functools

import jax
import jax.numpy as jnp
from jax import lax
from jax.experimental import pallas as pl
from jax.experimental.pallas import tpu as pltpu
from jax.experimental.pallas import tpu_sc as plsc

NC = 2
NS = 16
B = 128
CW = 32

_MESH = plsc.VectorSubcoreMesh(
    core_axis_name="c", subcore_axis_name="s", num_cores=NC, num_subcores=NS)


def _fill_zeros_1d(ref, n):
    def body(i, _):
        ref[pl.ds(i * 16, 16)] = jnp.zeros((16,), jnp.float32)
        return 0
    lax.fori_loop(0, n // 16, body, 0)


def _fill_zeros_2d(ref, nrows):
    def body(i, _):
        ref[i, pl.ds(0, 16)] = jnp.zeros((16,), jnp.float32)
        ref[i, pl.ds(16, 16)] = jnp.zeros((16,), jnp.float32)
        return 0
    lax.fori_loop(0, nrows, body, 0)


def _make_deg(NB, Npad):
    SP = Npad // NS

    @functools.partial(
        pl.kernel,
        out_type=(jax.ShapeDtypeStruct((Npad,), jnp.float32),
                  jax.ShapeDtypeStruct((Npad,), jnp.float32)),
        mesh=_MESH,
        compiler_params=pltpu.CompilerParams(use_tc_tiling_on_sc=False),
        scratch_types=[
            pltpu.VMEM((B,), jnp.int32),
            pltpu.VMEM((B,), jnp.float32),
            pltpu.VMEM((SP,), jnp.float32),
            pltpu.VMEM_SHARED((Npad,), jnp.float32),
        ],
    )
    def deg_kernel(dst_e, deg_a, deg_b, dst_v, ones_v, zer_v, acc_sh):
        c = lax.axis_index("c")
        t = lax.axis_index("s")
        w = c * NS + t
        for k in range(B // 16):
            ones_v[pl.ds(k * 16, 16)] = jnp.full((16,), 1.0, jnp.float32)
        _fill_zeros_1d(zer_v, SP)
        pltpu.sync_copy(zer_v, acc_sh.at[pl.ds(t * SP, SP)])
        plsc.subcore_barrier()
        nw = NC * NS
        nb = NB // nw + jnp.where(w < NB % nw, 1, 0)

        def body(i, _):
            j = w + i * nw
            pltpu.sync_copy(dst_e.at[pl.ds(j * B, B)], dst_v)
            pltpu.sync_copy(ones_v, acc_sh.at[dst_v], add=True)
            return 0

        lax.fori_loop(0, nb, body, 0)
        plsc.subcore_barrier()

        @pl.when(c == 0)
        def _():
            pltpu.sync_copy(acc_sh.at[pl.ds(t * SP, SP)],
                            deg_a.at[pl.ds(t * SP, SP)])

        @pl.when(c == 1)
        def _():
            pltpu.sync_copy(acc_sh.at[pl.ds(t * SP, SP)],
                            deg_b.at[pl.ds(t * SP, SP)])

    return deg_kernel


def _make_spmv(NB2, Npad):
    SP = Npad // NS
    nw = NC * NS
    TPW = NB2 // nw
    NCHV = TPW // _CHV
    MSV = (_CHV - NBUF) // NBUF

    @functools.partial(
        pl.kernel,
        out_type=(jax.ShapeDtypeStruct((Npad,), jnp.float32),
                  jax.ShapeDtypeStruct((Npad,), jnp.float32)),
        mesh=_MESH,
        compiler_params=pltpu.CompilerParams(use_tc_tiling_on_sc=False),
        scratch_types=[
            pltpu.VMEM((_CHV, B), jnp.int32),
            pltpu.VMEM((_CHV, B), jnp.int32),
            [pltpu.VMEM((B,), jnp.float32) for _ in range(NBUF)],
            pltpu.VMEM((SP,), jnp.float32),
            pltpu.VMEM_SHARED((Npad,), jnp.float32),
            [pltpu.SemaphoreType.DMA for _ in range(NBUF)],
        ],
    )
    def spmv_kernel(xp, src2, dst2, s_a, s_b,
                    srcc_v, dstc_v, vals_l, zer_v, acc_sh, sem_l):
        c = lax.axis_index("c")
        t = lax.axis_index("s")
        w = c * NS + t
        _fill_zeros_1d(zer_v, SP)
        pltpu.sync_copy(zer_v, acc_sh.at[pl.ds(t * SP, SP)])
        plsc.subcore_barrier()
        b0 = w * TPW

        def chunk_body(ci, _):
            cb = b0 + ci * _CHV
            pltpu.sync_copy(src2.at[pl.ds(cb, _CHV)], srcc_v)
            pltpu.sync_copy(dst2.at[pl.ds(cb, _CHV)], dstc_v)
            for q in range(NBUF):
                pltpu.async_copy(xp.at[srcc_v.at[q]], vals_l[q], sem_l[q])

            def ring_body(i2, _):
                for q in range(NBUF):
                    s = i2 * NBUF + q
                    pltpu.make_async_copy(
                        xp.at[srcc_v.at[s]], vals_l[q], sem_l[q]).wait()
                    pltpu.sync_copy(vals_l[q], acc_sh.at[dstc_v.at[s]],
                                    add=True)
                    pltpu.async_copy(xp.at[srcc_v.at[s + NBUF]],
                                     vals_l[q], sem_l[q])
                return 0

            lax.fori_loop(0, MSV, ring_body, 0)
            for q in range(NBUF):
                s = MSV * NBUF + q
                pltpu.make_async_copy(
                    xp.at[srcc_v.at[s]], vals_l[q], sem_l[q]).wait()
                pltpu.sync_copy(vals_l[q], acc_sh.at[dstc_v.at[s]],
                                add=True)
            return 0

        lax.fori_loop(0, NCHV, chunk_body, 0)
        plsc.subcore_barrier()

        @pl.when(c == 0)
        def _():
            pltpu.sync_copy(acc_sh.at[pl.ds(t * SP, SP)],
                            s_a.at[pl.ds(t * SP, SP)])

        @pl.when(c == 1)
        def _():
            pltpu.sync_copy(acc_sh.at[pl.ds(t * SP, SP)],
                            s_b.at[pl.ds(t * SP, SP)])

    return spmv_kernel


NBUF = 4
_CHV = 28
CH = 28
ZR = 184


def _make_spmm(NB2, Npad, HG, nacc):
    SPA = nacc // NS
    NZC = SPA // ZR
    GPC = HG // NC
    TPB = NB2 // NS
    NCH = TPB // CH
    MS = (CH - NBUF) // NBUF

    @functools.partial(
        pl.kernel,
        out_type=jax.ShapeDtypeStruct((HG, Npad, CW), jnp.float32),
        mesh=_MESH,
        compiler_params=pltpu.CompilerParams(use_tc_tiling_on_sc=False),
        scratch_types=[
            pltpu.VMEM((CH, B), jnp.int32),
            pltpu.VMEM((CH, B), jnp.int32),
            [pltpu.VMEM((B, CW), jnp.float32) for _ in range(NBUF)],
            pltpu.VMEM((ZR, CW), jnp.float32),
            pltpu.VMEM_SHARED((nacc, CW), jnp.float32),
            [pltpu.SemaphoreType.DMA for _ in range(NBUF)],
        ],
    )
    def spmm_kernel(z, src2, dst2, s_out,
                    srcc_v, dstc_v, rows_l, zer_v, acc_sh, sem_l):
        c = lax.axis_index("c")
        t = lax.axis_index("s")
        _fill_zeros_2d(zer_v, ZR)
        b0 = t * TPB
        for jg in range(GPC):
            g = c * GPC + jg
            off = g * Npad
            def zero_body(q4, _):
                pltpu.sync_copy(zer_v, acc_sh.at[pl.ds(t * SPA + q4 * ZR, ZR)])
                return 0
            lax.fori_loop(0, NZC, zero_body, 0)
            plsc.subcore_barrier()

            def chunk_body(ci, _):
                cb = b0 + ci * CH
                pltpu.sync_copy(src2.at[pl.ds(cb, CH)], srcc_v)
                pltpu.sync_copy(dst2.at[pl.ds(cb, CH)], dstc_v)

                def add_off(r, _):
                    for k in range(B // 16):
                        srcc_v[r, pl.ds(k * 16, 16)] = (
                            srcc_v[r, pl.ds(k * 16, 16)] + off)
                    return 0

                lax.fori_loop(0, CH, add_off, 0)
                for q in range(NBUF):
                    pltpu.async_copy(z.at[srcc_v.at[q]], rows_l[q], sem_l[q])

                def ring_body(i2, _):
                    for q in range(NBUF):
                        s = i2 * NBUF + q
                        pltpu.make_async_copy(
                            z.at[srcc_v.at[s]], rows_l[q], sem_l[q]).wait()
                        pltpu.sync_copy(rows_l[q], acc_sh.at[dstc_v.at[s]],
                                        add=True)
                        pltpu.async_copy(z.at[srcc_v.at[s + NBUF]],
                                         rows_l[q], sem_l[q])
                    return 0

                lax.fori_loop(0, MS, ring_body, 0)
                for q in range(NBUF):
                    s = MS * NBUF + q
                    pltpu.make_async_copy(
                        z.at[srcc_v.at[s]], rows_l[q], sem_l[q]).wait()
                    pltpu.sync_copy(rows_l[q], acc_sh.at[dstc_v.at[s]],
                                    add=True)
                return 0

            lax.fori_loop(0, NCH, chunk_body, 0)
            plsc.subcore_barrier()
            pltpu.sync_copy(acc_sh.at[pl.ds(t * SPA, SPA)],
                            s_out.at[g, pl.ds(t * SPA, SPA)])

    return spmm_kernel


def _make_l1_body(HG):
    def l1_body(s2a_ref, s2b_ref, xp_ref, dis_ref, w1_ref, b1_ref, w2_ref,
                out_ref, zf_ref):
        go = pl.program_id(1)

        @pl.when(go == 0)
        def _():
            sv = dis_ref[...] * (s2a_ref[...] + s2b_ref[...] + xp_ref[...])
            h = jnp.maximum(sv * w1_ref[...] + b1_ref[...], 0.0)
            zf = jnp.dot(h, w2_ref[...], preferred_element_type=jnp.float32)
            for gi in range(HG):
                zf_ref[gi] = zf[:, gi * CW:(gi + 1) * CW]

        out_ref[0] = zf_ref[go] * dis_ref[...]

    return l1_body


def _make_mid_body(HG):
    def mid_body(s_ref, z_ref, dis_ref, b_ref, w_ref, out_ref, hs_ref,
                 zf_ref):
        go = pl.program_id(1)

        @pl.when(go == 0)
        def _():
            dis = dis_ref[...]
            for gi in range(HG):
                hs_ref[:, pl.ds(gi * CW, CW)] = jnp.maximum(
                    dis * (s_ref[gi] + z_ref[gi]) + b_ref[gi], 0.0)
            zf = jnp.dot(hs_ref[...], w_ref[...],
                         preferred_element_type=jnp.float32)
            for gi in range(HG):
                zf_ref[gi] = zf[:, gi * CW:(gi + 1) * CW]

        out_ref[0] = zf_ref[go] * dis_ref[...]

    return mid_body


def _make_fin_body(R, NBLK, HG, n_real):
    def fin_body(s_ref, z_ref, dis_ref, b_ref, f1w_ref, f1b_ref, f2w_ref,
                 f2b_ref, out_ref, acc_ref):
        i = pl.program_id(0)
        dis = dis_ref[...]
        rows = i * R + lax.broadcasted_iota(jnp.int32, (R, 1), 0)
        mask = rows < n_real
        for gi in range(HG):
            h = jnp.maximum(dis * (s_ref[gi] + z_ref[gi]) + b_ref[gi], 0.0)
            h = jnp.where(mask, h, 0.0)
            part = jnp.sum(h, axis=0, keepdims=True)

            @pl.when(i == 0)
            def _():
                acc_ref[:, pl.ds(gi * CW, CW)] = part

            @pl.when(i > 0)
            def _():
                acc_ref[:, pl.ds(gi * CW, CW)] = (
                    acc_ref[:, pl.ds(gi * CW, CW)] + part)

        @pl.when(i == NBLK - 1)
        def _():
            g = acc_ref[...] * (1.0 / n_real)
            o = jnp.maximum(
                jnp.dot(g, f1w_ref[...], preferred_element_type=jnp.float32)
                + f1b_ref[...], 0.0)
            out_ref[...] = (jnp.dot(o, f2w_ref[...],
                                    preferred_element_type=jnp.float32)
                            + f2b_ref[...])

    return fin_body


def _prep_body(dega_ref, degb_ref, x_ref, dis_ref, xp_ref):
    d = dega_ref[...] + degb_ref[...] + 1.0
    dis = lax.rsqrt(d)
    dis_ref[...] = dis
    xp_ref[...] = dis * x_ref[...]


def kernel(x, edge_index, batch, W1, b1, W2, b2, W3, b3, W4, b4,
           f1W, f1b, f2W, f2b):
    N = x.shape[0]
    E = edge_index.shape[1]
    H = W2.shape[0]
    HG = H // CW
    Npad = ((N + NS * B - 1) // (NS * B)) * (NS * B)
    NB = E // B
    R = 1600
    NBLK = Npad // R

    nacc = ((N + NS * 8 - 1) // (NS * 8)) * (NS * 8)
    nw = NC * NS
    NB2 = ((NB + nw - 1) // nw) * nw
    EP = NB2 * B - E
    src_e = jnp.concatenate(
        [edge_index[0], (jnp.arange(EP, dtype=jnp.int32) % N)])
    dst_e = jnp.concatenate(
        [edge_index[1], N + (jnp.arange(EP, dtype=jnp.int32) % (nacc - N))])
    src2 = src_e.reshape(NB2, B)
    dst2 = dst_e.reshape(NB2, B)
    x_pad = jnp.concatenate(
        [x, jnp.zeros((Npad - N, 1), jnp.float32)], axis=0)

    deg_a, deg_b = _make_deg(NB2, Npad)(dst_e)

    vspec1 = pl.BlockSpec((R, 1), lambda i: (i, 0))
    dis, xp = pl.pallas_call(
        _prep_body,
        grid=(NBLK,),
        in_specs=[vspec1, vspec1, vspec1],
        out_specs=(vspec1, vspec1),
        out_shape=(jax.ShapeDtypeStruct((Npad, 1), jnp.float32),
                   jax.ShapeDtypeStruct((Npad, 1), jnp.float32)),
    )(deg_a.reshape(Npad, 1), deg_b.reshape(Npad, 1), x_pad)

    s2a, s2b = _make_spmv(NB2, Npad)(xp.reshape(Npad), src2, dst2)

    grp_spec = pl.BlockSpec((HG, R, CW), lambda i, go: (0, i, 0))
    vec_spec2 = pl.BlockSpec((R, 1), lambda i, go: (i, 0))
    b8_spec = pl.BlockSpec((HG, 1, CW), lambda i, go: (0, 0, 0))
    out_spec = pl.BlockSpec((1, R, CW), lambda i, go: (go, i, 0))
    zg_shape = jax.ShapeDtypeStruct((HG, Npad, CW), jnp.float32)

    def split_b(b):
        return b.reshape(HG, 1, CW)

    wfull_spec = pl.BlockSpec((H, H), lambda i, go: (0, 0))
    Z2 = pl.pallas_call(
        _make_l1_body(HG),
        grid=(NBLK, HG),
        in_specs=[vec_spec2, vec_spec2, vec_spec2, vec_spec2,
                  pl.BlockSpec((1, H), lambda i, go: (0, 0)),
                  pl.BlockSpec((1, H), lambda i, go: (0, 0)),
                  wfull_spec],
        out_specs=out_spec,
        out_shape=zg_shape,
        scratch_shapes=[pltpu.VMEM((HG, R, CW), jnp.float32)],
    )(s2a.reshape(Npad, 1), s2b.reshape(Npad, 1), xp, dis,
      W1, b1.reshape(1, H), W2)

    spmm = _make_spmm(NB2, Npad, HG, nacc)

    def mid(S, Z, bprev, Wnext):
        return pl.pallas_call(
            _make_mid_body(HG),
            grid=(NBLK, HG),
            in_specs=[grp_spec, grp_spec, vec_spec2, b8_spec, wfull_spec],
            out_specs=out_spec,
            out_shape=zg_shape,
            scratch_shapes=[pltpu.VMEM((R, H), jnp.float32),
                            pltpu.VMEM((HG, R, CW), jnp.float32)],
        )(S, Z, dis, split_b(bprev), Wnext)

    S2 = spmm(Z2.reshape(HG * Npad, CW), src2, dst2)
    Z3 = mid(S2, Z2, b2, W3)
    S3 = spmm(Z3.reshape(HG * Npad, CW), src2, dst2)
    Z4 = mid(S3, Z3, b3, W4)
    S4 = spmm(Z4.reshape(HG * Npad, CW), src2, dst2)

    y = pl.pallas_call(
        _make_fin_body(R, NBLK, HG, N),
        grid=(NBLK,),
        in_specs=[pl.BlockSpec((HG, R, CW), lambda i: (0, i, 0)),
                  pl.BlockSpec((HG, R, CW), lambda i: (0, i, 0)),
                  pl.BlockSpec((R, 1), lambda i: (i, 0)),
                  pl.BlockSpec((HG, 1, CW), lambda i: (0, 0, 0)),
                  pl.BlockSpec((H, H), lambda i: (0, 0)),
                  pl.BlockSpec((1, H), lambda i: (0, 0)),
                  pl.BlockSpec((H, 1), lambda i: (0, 0)),
                  pl.BlockSpec((1, 1), lambda i: (0, 0))],
        out_specs=pl.BlockSpec((1, 1), lambda i: (0, 0)),
        out_shape=jax.ShapeDtypeStruct((1, 1), jnp.float32),
        scratch_shapes=[pltpu.VMEM((1, H), jnp.float32)],
    )(S4, Z4, dis, split_b(b4), f1W, f1b.reshape(1, H),
      f2W, f2b.reshape(1, 1))

    return y
